# Initial kernel scaffold; baseline (speedup 1.0000x reference)
import functools

import jax
import jax.numpy as jnp
import numpy as np
from jax import lax
from jax.experimental import pallas as pl
from jax.experimental.pallas import tpu as pltpu
from jax.experimental.pallas import tpu_sc as plsc

N = 10000
E = 320000
F = 128
H = 128
TW = 144
NC = 2
NS = 16
NW = NC * NS
G = 128
NG = E // G
ROWS_PER_TILE = N // NS
ZR = 125
NB = 2000
EBLK = 1280

_silu = jax.nn.silu


def _prep_body(h_ref, ws_ref, wr_ref, p_ref, ts_ref, tr_ref):
    h = h_ref[...]
    p = p_ref[...]
    a = jnp.dot(h, ws_ref[...], preferred_element_type=jnp.float32)
    r = jnp.dot(h, wr_ref[...], preferred_element_type=jnp.float32)
    ts_ref[...] = jnp.concatenate([a, p], axis=-1)
    tr_ref[...] = jnp.concatenate([r, p], axis=-1)


def _prep_tables(h, ws, wr, pos16):
    return pl.pallas_call(
        _prep_body,
        grid=(N // NB,),
        in_specs=[
            pl.BlockSpec((NB, F), lambda i: (i, 0)),
            pl.BlockSpec((F, H), lambda i: (0, 0)),
            pl.BlockSpec((F, H), lambda i: (0, 0)),
            pl.BlockSpec((NB, 16), lambda i: (i, 0)),
        ],
        out_specs=[
            pl.BlockSpec((NB, TW), lambda i: (i, 0)),
            pl.BlockSpec((NB, TW), lambda i: (i, 0)),
        ],
        out_shape=[
            jax.ShapeDtypeStruct((N, TW), jnp.float32),
            jax.ShapeDtypeStruct((N, TW), jnp.float32),
        ],
    )(h, ws, wr, pos16)


def _gather_kernel(ts_hbm, tr_hbm, snd_hbm, rcv_hbm, gs_hbm, gr_hbm,
                   idx_v, rows_v, sem):
    wid = lax.axis_index("s") * NC + lax.axis_index("c")
    ngroups = 78 + jnp.where(wid < NG - 78 * NW, 1, 0)

    def body(i, _):
        g = wid + i * NW
        off = g * G
        pltpu.sync_copy(snd_hbm.at[pl.ds(off, G)], idx_v)
        pltpu.async_copy(ts_hbm.at[idx_v], rows_v, sem).wait()
        pltpu.sync_copy(rows_v, gs_hbm.at[pl.ds(off, G)])
        pltpu.sync_copy(rcv_hbm.at[pl.ds(off, G)], idx_v)
        pltpu.async_copy(tr_hbm.at[idx_v], rows_v, sem).wait()
        pltpu.sync_copy(rows_v, gr_hbm.at[pl.ds(off, G)])
        return 0

    lax.fori_loop(0, ngroups, body, 0)


def _gather(ts, tr, senders, receivers):
    mesh = plsc.VectorSubcoreMesh(core_axis_name="c", subcore_axis_name="s")
    f = functools.partial(
        pl.kernel,
        out_type=(
            jax.ShapeDtypeStruct((E, TW), jnp.float32),
            jax.ShapeDtypeStruct((E, TW), jnp.float32),
        ),
        mesh=mesh,
        scratch_types=[
            pltpu.VMEM((G,), jnp.int32),
            pltpu.VMEM((G, TW), jnp.float32),
            pltpu.SemaphoreType.DMA,
        ],
    )(_gather_kernel)
    return f(ts, tr, senders, receivers)


def _edge_body(gs_ref, gr_ref, b0_ref, wl0_ref, w1_ref, b1_ref,
               xtw0_ref, xtb0_ref, xtw1_ref, xtb1_ref, xow_ref, xob_ref,
               ew_ref, eb_ref, out_ref):
    gs = gs_ref[...]
    gr = gr_ref[...]
    vec = gr[:, F:TW] - gs[:, F:TW]
    x2 = jnp.sum(vec * vec, axis=-1, keepdims=True)
    ln = jnp.where(x2 == 0.0, 0.0,
                   jnp.sqrt(jnp.where(x2 == 0.0, 1.0, x2)))
    t0 = _silu(gs[:, :F] + gr[:, :F] + ln * wl0_ref[...] + b0_ref[...])
    m1 = _silu(jnp.dot(t0, w1_ref[...],
                       preferred_element_type=jnp.float32) + b1_ref[...])
    p1 = _silu(jnp.dot(m1, xtw0_ref[...],
                       preferred_element_type=jnp.float32) + xtb0_ref[...])
    p2 = _silu(jnp.dot(p1, xtw1_ref[...],
                       preferred_element_type=jnp.float32) + xtb1_ref[...])
    phx = jnp.sum(p2 * xow_ref[...], axis=-1, keepdims=True) + xob_ref[0]
    e = jax.nn.sigmoid(jnp.sum(m1 * ew_ref[...], axis=-1, keepdims=True)
                       + eb_ref[0])
    msg = m1 * e
    shift = phx * vec / (1.0 + ln)
    out_ref[...] = jnp.concatenate([msg, shift], axis=-1)


def _edge_mlp(gs, gr, b0, wl0, w1, b1, xtw0, xtb0, xtw1, xtb1,
              xow, xob, ew, eb):
    vec_spec = pl.BlockSpec((H,), lambda i: (0,))
    mat_spec = pl.BlockSpec((H, H), lambda i: (0, 0))
    one_spec = pl.BlockSpec((1,), lambda i: (0,))
    return pl.pallas_call(
        _edge_body,
        grid=(E // EBLK,),
        in_specs=[
            pl.BlockSpec((EBLK, TW), lambda i: (i, 0)),
            pl.BlockSpec((EBLK, TW), lambda i: (i, 0)),
            vec_spec, vec_spec, mat_spec, vec_spec,
            mat_spec, vec_spec, mat_spec, vec_spec,
            vec_spec, one_spec, vec_spec, one_spec,
        ],
        out_specs=pl.BlockSpec((EBLK, TW), lambda i: (i, 0)),
        out_shape=jax.ShapeDtypeStruct((E, TW), jnp.float32),
    )(gs, gr, b0, wl0, w1, b1, xtw0, xtb0, xtw1, xtb1, xow, xob, ew, eb)


def _scatter_kernel(e_hbm, rcv_hbm, out_hbm, idx_v, rows_v, zbuf_v, acc_sh):
    cid = lax.axis_index("c")
    sid = lax.axis_index("s")
    wid = sid * NC + cid

    def zrow(i, _):
        r = i // 9
        c = i % 9
        zbuf_v[r, pl.ds(c * 16, 16)] = jnp.zeros((16,), jnp.float32)
        return 0
    lax.fori_loop(0, ZR * 9, zrow, 0)

    tbase = sid * ROWS_PER_TILE

    def zcp(j, _):
        pltpu.sync_copy(zbuf_v, acc_sh.at[pl.ds(tbase + j * ZR, ZR)])
        return 0
    lax.fori_loop(0, ROWS_PER_TILE // ZR, zcp, 0)
    plsc.subcore_barrier()

    ngroups = 78 + jnp.where(wid < NG - 78 * NW, 1, 0)

    def body(i, _):
        off = (wid + i * NW) * G
        pltpu.sync_copy(rcv_hbm.at[pl.ds(off, G)], idx_v)
        pltpu.sync_copy(e_hbm.at[pl.ds(off, G)], rows_v)
        pltpu.sync_copy(rows_v, acc_sh.at[idx_v], add=True)
        return 0

    lax.fori_loop(0, ngroups, body, 0)
    plsc.subcore_barrier()

    def wcp(j, _):
        r0 = tbase + j * ZR
        pltpu.sync_copy(acc_sh.at[pl.ds(r0, ZR)], zbuf_v)
        pltpu.sync_copy(zbuf_v, out_hbm.at[cid, pl.ds(r0, ZR)])
        return 0
    lax.fori_loop(0, ROWS_PER_TILE // ZR, wcp, 0)


def _scatter(edge_out, receivers):
    mesh = plsc.VectorSubcoreMesh(core_axis_name="c", subcore_axis_name="s")
    f = functools.partial(
        pl.kernel,
        out_type=jax.ShapeDtypeStruct((NC, N, TW), jnp.float32),
        mesh=mesh,
        scratch_types=[
            pltpu.VMEM((G,), jnp.int32),
            pltpu.VMEM((G, TW), jnp.float32),
            pltpu.VMEM((ZR, TW), jnp.float32),
            pltpu.VMEM_SHARED((N, TW), jnp.float32),
        ],
    )(_scatter_kernel)
    return f(edge_out, receivers)


def _node_body(p0_ref, p1_ref, h_ref, pos_ref, hw0m_ref, hw0h_ref, hb0_ref,
               hw1_ref, hb1_ref, hw2_ref, hb2_ref, vout_ref, fout_ref):
    ps = p0_ref[...] + p1_ref[...]
    hf = h_ref[...]
    m_i = ps[:, :F] / np.float32(np.sqrt(9999.0))
    shift = ps[:, F:TW] / 9999.0
    vout_ref[...] = pos_ref[...] + shift[:, :3]
    t = _silu(jnp.dot(m_i, hw0m_ref[...], preferred_element_type=jnp.float32)
              + jnp.dot(hf, hw0h_ref[...], preferred_element_type=jnp.float32)
              + hb0_ref[...])
    t = _silu(jnp.dot(t, hw1_ref[...], preferred_element_type=jnp.float32)
              + hb1_ref[...])
    fout_ref[...] = (jnp.dot(t, hw2_ref[...],
                             preferred_element_type=jnp.float32)
                     + hb2_ref[...] + hf)


def _node_mlp(p0, p1, h, pos, hw0m, hw0h, hb0, hw1, hb1, hw2, hb2):
    vec_spec = pl.BlockSpec((H,), lambda i: (0,))
    mat_spec = pl.BlockSpec((H, H), lambda i: (0, 0))
    return pl.pallas_call(
        _node_body,
        grid=(N // NB,),
        in_specs=[
            pl.BlockSpec((NB, TW), lambda i: (i, 0)),
            pl.BlockSpec((NB, TW), lambda i: (i, 0)),
            pl.BlockSpec((NB, F), lambda i: (i, 0)),
            pl.BlockSpec((NB, 3), lambda i: (i, 0)),
            mat_spec, mat_spec, vec_spec,
            mat_spec, vec_spec, mat_spec, vec_spec,
        ],
        out_specs=[
            pl.BlockSpec((NB, 3), lambda i: (i, 0)),
            pl.BlockSpec((NB, F), lambda i: (i, 0)),
        ],
        out_shape=[
            jax.ShapeDtypeStruct((N, 3), jnp.float32),
            jax.ShapeDtypeStruct((N, F), jnp.float32),
        ],
    )(p0, p1, h, pos, hw0m, hw0h, hb0, hw1, hb1, hw2, hb2)


def kernel(node_positions, node_features, senders, receivers,
           phi_e_w0, phi_e_b0, phi_e_w1, phi_e_b1,
           phi_xt_w0, phi_xt_b0, phi_xt_w1, phi_xt_b1,
           phi_x_out_w, phi_x_out_b, e_w, e_b,
           phi_h_w0, phi_h_b0, phi_h_w1, phi_h_b1, phi_h_w2, phi_h_b2):
    pos16 = jnp.pad(node_positions, ((0, 0), (0, 13)))
    ws = phi_e_w0[:F]
    wr = phi_e_w0[F:2 * F]
    wl0 = phi_e_w0[2 * F]
    xow = phi_x_out_w[:, 0]
    ew = e_w[:, 0]
    hw0m = phi_h_w0[:F]
    hw0h = phi_h_w0[F:]

    ts, tr = _prep_tables(node_features, ws, wr, pos16)
    gs, gr = _gather(ts, tr, senders, receivers)
    edge_out = _edge_mlp(gs, gr, phi_e_b0, wl0, phi_e_w1, phi_e_b1,
                         phi_xt_w0, phi_xt_b0, phi_xt_w1, phi_xt_b1,
                         xow, phi_x_out_b, ew, e_b)
    part = _scatter(edge_out, receivers)
    p0 = part[0]
    p1 = part[1]
    vectors_out, features_out = _node_mlp(
        p0, p1, node_features, node_positions,
        hw0m, hw0h, phi_h_b0, phi_h_w1, phi_h_b1, phi_h_w2, phi_h_b2)
    return (vectors_out, features_out)


# speedup vs baseline: 3.4876x; 3.4876x over previous
import functools

import jax
import jax.numpy as jnp
import numpy as np
from jax import lax
from jax.experimental import pallas as pl
from jax.experimental.pallas import tpu as pltpu
from jax.experimental.pallas import tpu_sc as plsc

N = 10000
E = 320000
F = 128
H = 128
PW = 16
NC = 2
NS = 16
NW = NC * NS
G = 128
NG = E // G
NG_BASE = NG // NW
NG_REM = NG - NG_BASE * NW
RB = 624
CW = 16
NB = 2000
EBLK = 1280

_silu = jax.nn.silu


def _worker_id():
    return lax.axis_index("s") * NC + lax.axis_index("c")


def _ngroups(wid):
    return NG_BASE + jnp.where(wid < NG_REM, 1, 0)


def _prep_body(h_ref, ws_ref, wr_ref, p_ref, ts_ref, tr_ref, pp_ref, pn_ref):
    h = h_ref[...]
    p = p_ref[...]
    ts_ref[...] = jnp.dot(h, ws_ref[...], preferred_element_type=jnp.float32)
    tr_ref[...] = jnp.dot(h, wr_ref[...], preferred_element_type=jnp.float32)
    pp_ref[...] = p
    pn_ref[...] = -p


def _prep_tables(h, ws, wr, pos16):
    return pl.pallas_call(
        _prep_body,
        grid=(N // NB,),
        in_specs=[
            pl.BlockSpec((NB, F), lambda i: (i, 0)),
            pl.BlockSpec((F, H), lambda i: (0, 0)),
            pl.BlockSpec((F, H), lambda i: (0, 0)),
            pl.BlockSpec((NB, PW), lambda i: (i, 0)),
        ],
        out_specs=[
            pl.BlockSpec((NB, F), lambda i: (i, 0)),
            pl.BlockSpec((NB, F), lambda i: (i, 0)),
            pl.BlockSpec((NB, PW), lambda i: (i, 0)),
            pl.BlockSpec((NB, PW), lambda i: (i, 0)),
        ],
        out_shape=[
            jax.ShapeDtypeStruct((N, F), jnp.float32),
            jax.ShapeDtypeStruct((N, F), jnp.float32),
            jax.ShapeDtypeStruct((N, PW), jnp.float32),
            jax.ShapeDtypeStruct((N, PW), jnp.float32),
        ],
    )(h, ws, wr, pos16)


def _make_gather(width, tc_tiling):
    def gather_kernel(ta_hbm, tb_hbm, snd_hbm, rcv_hbm, out_hbm,
                      idx_v, rows_v, sem):
        wid = _worker_id()

        def body(i, _):
            off = (wid + i * NW) * G
            pltpu.sync_copy(snd_hbm.at[pl.ds(off, G)], idx_v)
            pltpu.async_copy(ta_hbm.at[idx_v], rows_v, sem).wait()
            pltpu.sync_copy(rcv_hbm.at[pl.ds(off, G)], idx_v)
            pltpu.async_copy(tb_hbm.at[idx_v], rows_v, sem, add=True).wait()
            pltpu.sync_copy(rows_v, out_hbm.at[pl.ds(off, G)])
            return 0

        lax.fori_loop(0, _ngroups(wid), body, 0)

    mesh = plsc.VectorSubcoreMesh(core_axis_name="c", subcore_axis_name="s")
    return functools.partial(
        pl.kernel,
        out_type=jax.ShapeDtypeStruct((E, width), jnp.float32),
        mesh=mesh,
        scratch_types=[
            pltpu.VMEM((G,), jnp.int32),
            pltpu.VMEM((G, width), jnp.float32),
            pltpu.SemaphoreType.DMA,
        ],
        compiler_params=pltpu.CompilerParams(use_tc_tiling_on_sc=tc_tiling),
    )(gather_kernel)


def _edge_body(es_ref, vec_ref, b0_ref, wl0_ref, w1_ref, b1_ref,
               xtw0_ref, xtb0_ref, xtw1_ref, xtb1_ref, xow_ref, xob_ref,
               ew_ref, eb_ref, msg_ref, sh_ref):
    vec = vec_ref[...]
    x2 = jnp.sum(vec * vec, axis=-1, keepdims=True)
    ln = jnp.where(x2 == 0.0, 0.0,
                   jnp.sqrt(jnp.where(x2 == 0.0, 1.0, x2)))
    t0 = _silu(es_ref[...] + ln * wl0_ref[...] + b0_ref[...])
    m1 = _silu(jnp.dot(t0, w1_ref[...],
                       preferred_element_type=jnp.float32) + b1_ref[...])
    p1 = _silu(jnp.dot(m1, xtw0_ref[...],
                       preferred_element_type=jnp.float32) + xtb0_ref[...])
    p2 = _silu(jnp.dot(p1, xtw1_ref[...],
                       preferred_element_type=jnp.float32) + xtb1_ref[...])
    phx = jnp.sum(p2 * xow_ref[...], axis=-1, keepdims=True) + xob_ref[0]
    e = jax.nn.sigmoid(jnp.sum(m1 * ew_ref[...], axis=-1, keepdims=True)
                       + eb_ref[0])
    msg_ref[...] = m1 * e
    sh_ref[...] = phx * vec / (1.0 + ln)


def _edge_mlp(esum, vec, b0, wl0, w1, b1, xtw0, xtb0, xtw1, xtb1,
              xow, xob, ew, eb):
    vec_spec = pl.BlockSpec((H,), lambda i: (0,))
    mat_spec = pl.BlockSpec((H, H), lambda i: (0, 0))
    one_spec = pl.BlockSpec((1,), lambda i: (0,))
    return pl.pallas_call(
        _edge_body,
        grid=(E // EBLK,),
        in_specs=[
            pl.BlockSpec((EBLK, F), lambda i: (i, 0)),
            pl.BlockSpec((EBLK, PW), lambda i: (i, 0)),
            vec_spec, vec_spec, mat_spec, vec_spec,
            mat_spec, vec_spec, mat_spec, vec_spec,
            vec_spec, one_spec, vec_spec, one_spec,
        ],
        out_specs=[
            pl.BlockSpec((EBLK, F), lambda i: (i, 0)),
            pl.BlockSpec((EBLK, PW), lambda i: (i, 0)),
        ],
        out_shape=[
            jax.ShapeDtypeStruct((E, F), jnp.float32),
            jax.ShapeDtypeStruct((E, PW), jnp.float32),
        ],
    )(esum, vec, b0, wl0, w1, b1, xtw0, xtb0, xtw1, xtb1, xow, xob, ew, eb)


def _make_scatter(width, tc_tiling):
    nlanes = width // 16

    def scatter_kernel(e_hbm, rcv_hbm, out_hbm, idx_v, rows_v, zbuf_v,
                       acc_sh):
        cid = lax.axis_index("c")
        sid = lax.axis_index("s")
        wid = sid * NC + cid

        def zrow(i, _):
            r = i // nlanes
            c = i % nlanes
            zbuf_v[r, pl.ds(c * 16, 16)] = jnp.zeros((16,), jnp.float32)
            return 0
        lax.fori_loop(0, CW * nlanes, zrow, 0)

        tbase = sid * RB
        ncp = (RB + jnp.where(sid == NS - 1, N - NS * RB, 0)) // CW

        def zcp(j, _):
            pltpu.sync_copy(zbuf_v, acc_sh.at[pl.ds(tbase + j * CW, CW)])
            return 0
        lax.fori_loop(0, ncp, zcp, 0)
        plsc.subcore_barrier()

        def body(i, _):
            off = (wid + i * NW) * G
            pltpu.sync_copy(rcv_hbm.at[pl.ds(off, G)], idx_v)
            pltpu.sync_copy(e_hbm.at[pl.ds(off, G)], rows_v)
            pltpu.sync_copy(rows_v, acc_sh.at[idx_v], add=True)
            return 0

        lax.fori_loop(0, _ngroups(wid), body, 0)
        plsc.subcore_barrier()

        def wcp(j, _):
            r0 = tbase + j * CW
            pltpu.sync_copy(acc_sh.at[pl.ds(r0, CW)], zbuf_v)
            pltpu.sync_copy(zbuf_v, out_hbm.at[cid, pl.ds(r0, CW)])
            return 0
        lax.fori_loop(0, ncp, wcp, 0)

    mesh = plsc.VectorSubcoreMesh(core_axis_name="c", subcore_axis_name="s")
    return functools.partial(
        pl.kernel,
        out_type=jax.ShapeDtypeStruct((NC, N, width), jnp.float32),
        mesh=mesh,
        scratch_types=[
            pltpu.VMEM((G,), jnp.int32),
            pltpu.VMEM((G, width), jnp.float32),
            pltpu.VMEM((CW, width), jnp.float32),
            pltpu.VMEM_SHARED((N, width), jnp.float32),
        ],
        compiler_params=pltpu.CompilerParams(use_tc_tiling_on_sc=tc_tiling),
    )(scatter_kernel)


def _node_body(pm_ref, psh_ref, h_ref, pos_ref, hw0m_ref, hw0h_ref, hb0_ref,
               hw1_ref, hb1_ref, hw2_ref, hb2_ref, vout_ref, fout_ref):
    hf = h_ref[...]
    m_i = (pm_ref[0] + pm_ref[1]) / np.float32(np.sqrt(9999.0))
    shift = (psh_ref[0] + psh_ref[1]) / 9999.0
    vout_ref[...] = pos_ref[...] + shift[:, :3]
    t = _silu(jnp.dot(m_i, hw0m_ref[...], preferred_element_type=jnp.float32)
              + jnp.dot(hf, hw0h_ref[...], preferred_element_type=jnp.float32)
              + hb0_ref[...])
    t = _silu(jnp.dot(t, hw1_ref[...], preferred_element_type=jnp.float32)
              + hb1_ref[...])
    fout_ref[...] = (jnp.dot(t, hw2_ref[...],
                             preferred_element_type=jnp.float32)
                     + hb2_ref[...] + hf)


def _node_mlp(pm, psh, h, pos, hw0m, hw0h, hb0, hw1, hb1, hw2, hb2):
    vec_spec = pl.BlockSpec((H,), lambda i: (0,))
    mat_spec = pl.BlockSpec((H, H), lambda i: (0, 0))
    return pl.pallas_call(
        _node_body,
        grid=(N // NB,),
        in_specs=[
            pl.BlockSpec((NC, NB, F), lambda i: (0, i, 0)),
            pl.BlockSpec((NC, NB, PW), lambda i: (0, i, 0)),
            pl.BlockSpec((NB, F), lambda i: (i, 0)),
            pl.BlockSpec((NB, 3), lambda i: (i, 0)),
            mat_spec, mat_spec, vec_spec,
            mat_spec, vec_spec, mat_spec, vec_spec,
        ],
        out_specs=[
            pl.BlockSpec((NB, 3), lambda i: (i, 0)),
            pl.BlockSpec((NB, F), lambda i: (i, 0)),
        ],
        out_shape=[
            jax.ShapeDtypeStruct((N, 3), jnp.float32),
            jax.ShapeDtypeStruct((N, F), jnp.float32),
        ],
    )(pm, psh, h, pos, hw0m, hw0h, hb0, hw1, hb1, hw2, hb2)


def kernel(node_positions, node_features, senders, receivers,
           phi_e_w0, phi_e_b0, phi_e_w1, phi_e_b1,
           phi_xt_w0, phi_xt_b0, phi_xt_w1, phi_xt_b1,
           phi_x_out_w, phi_x_out_b, e_w, e_b,
           phi_h_w0, phi_h_b0, phi_h_w1, phi_h_b1, phi_h_w2, phi_h_b2):
    pos16 = jnp.pad(node_positions, ((0, 0), (0, PW - 3)))
    ws = phi_e_w0[:F]
    wr = phi_e_w0[F:2 * F]
    wl0 = phi_e_w0[2 * F]
    xow = phi_x_out_w[:, 0]
    ew = e_w[:, 0]
    hw0m = phi_h_w0[:F]
    hw0h = phi_h_w0[F:]

    ts, tr, pp, pn = _prep_tables(node_features, ws, wr, pos16)
    esum = _make_gather(F, True)(ts, tr, senders, receivers)
    vec = _make_gather(PW, False)(pn, pp, senders, receivers)
    msg, sh = _edge_mlp(esum, vec, phi_e_b0, wl0, phi_e_w1, phi_e_b1,
                        phi_xt_w0, phi_xt_b0, phi_xt_w1, phi_xt_b1,
                        xow, phi_x_out_b, ew, e_b)
    pm = _make_scatter(F, True)(msg, receivers)
    psh = _make_scatter(PW, False)(sh, receivers)
    vectors_out, features_out = _node_mlp(
        pm, psh, node_features, node_positions,
        hw0m, hw0h, phi_h_b0, phi_h_w1, phi_h_b1, phi_h_w2, phi_h_b2)
    return (vectors_out, features_out)


# speedup vs baseline: 5.0956x; 1.4611x over previous
import functools

import jax
import jax.numpy as jnp
import numpy as np
from jax import lax
from jax.experimental import pallas as pl
from jax.experimental.pallas import tpu as pltpu
from jax.experimental.pallas import tpu_sc as plsc

N = 10000
E = 320000
F = 128
H = 128
PW = 16
NC = 2
NS = 16
NW = NC * NS
G = 128
NCH = 2
EC = E // NCH
NGC = EC // G
CBASE = NGC // NW
CREM = NGC - CBASE * NW
CGMAX = CBASE + 1
CGWIN = 48
CPAD = 1280
NBUF = 3
CNBLK = CBASE // NBUF
SBUF = 2
CSBLK = CBASE // SBUF
RB = 624
CW = 16
NB = 2000
EBLK = 1280


def _sigm(x):
    return 0.5 * jnp.tanh(0.5 * x) + 0.5


def _silu(x):
    return x * _sigm(x)


def _worker_id():
    return lax.axis_index("s") * NC + lax.axis_index("c")


def _worker_span(wid):
    gstart = wid * CBASE + jnp.minimum(wid, CREM)
    ng = CBASE + jnp.where(wid < CREM, 1, 0)
    return gstart, ng


def _load_idx(idx2d_hbm, gstart, idx_v):
    gsa = (gstart // 8) * 8
    pltpu.sync_copy(idx2d_hbm.at[pl.ds(gsa, CGWIN)], idx_v)
    return gstart - gsa


def _prep_body(h_ref, ws_ref, wr_ref, p_ref, ts_ref, tr_ref, pp_ref, pn_ref):
    h = h_ref[...]
    p = p_ref[...]
    ts_ref[...] = jnp.dot(h, ws_ref[...], preferred_element_type=jnp.float32)
    tr_ref[...] = jnp.dot(h, wr_ref[...], preferred_element_type=jnp.float32)
    pp_ref[...] = p
    pn_ref[...] = -p


def _prep_tables(h, ws, wr, pos16):
    return pl.pallas_call(
        _prep_body,
        grid=(N // NB,),
        in_specs=[
            pl.BlockSpec((NB, F), lambda i: (i, 0)),
            pl.BlockSpec((F, H), lambda i: (0, 0)),
            pl.BlockSpec((F, H), lambda i: (0, 0)),
            pl.BlockSpec((NB, PW), lambda i: (i, 0)),
        ],
        out_specs=[
            pl.BlockSpec((NB, F), lambda i: (i, 0)),
            pl.BlockSpec((NB, F), lambda i: (i, 0)),
            pl.BlockSpec((NB, PW), lambda i: (i, 0)),
            pl.BlockSpec((NB, PW), lambda i: (i, 0)),
        ],
        out_shape=[
            jax.ShapeDtypeStruct((N, F), jnp.float32),
            jax.ShapeDtypeStruct((N, F), jnp.float32),
            jax.ShapeDtypeStruct((N, PW), jnp.float32),
            jax.ShapeDtypeStruct((N, PW), jnp.float32),
        ],
    )(h, ws, wr, pos16)


def _make_gather(width, tc_tiling):
    def gather_kernel(ta_hbm, tb_hbm, snd2d_hbm, rcv2d_hbm, out_hbm,
                      idxs_v, idxr_v, rows_v, sem0, sem1, sem2):
        wid = _worker_id()
        gstart, ng = _worker_span(wid)
        d0 = _load_idx(snd2d_hbm, gstart, idxs_v)
        _load_idx(rcv2d_hbm, gstart, idxr_v)
        sems = (sem0, sem1, sem2)

        def fire_gather(g, b):
            pltpu.async_copy(ta_hbm.at[idxs_v.at[d0 + g]], rows_v.at[b],
                             sems[b])

        def fire_add(g, b):
            pltpu.async_copy(tb_hbm.at[idxr_v.at[d0 + g]], rows_v.at[b],
                             sems[b], add=True)

        def fire_store(g, b):
            pltpu.async_copy(rows_v.at[b],
                             out_hbm.at[pl.ds((gstart + g) * G, G)], sems[b])

        def wait(b):
            pltpu.make_async_copy(out_hbm.at[pl.ds(gstart * G, G)],
                                  rows_v.at[b], sems[b]).wait()

        def blk_body(blk, _):
            g0 = blk * NBUF
            for b in range(NBUF):
                @pl.when(blk > 0)
                def _(b=b):
                    wait(b)
                fire_gather(g0 + b, b)
            for b in range(NBUF):
                wait(b)
                fire_add(g0 + b, b)
            for b in range(NBUF):
                wait(b)
                fire_store(g0 + b, b)
            return 0

        lax.fori_loop(0, CNBLK, blk_body, 0)
        for b in range(NBUF):
            wait(b)

        @pl.when(ng == CGMAX)
        def _():
            g = CBASE
            fire_gather(g, 0)
            wait(0)
            fire_add(g, 0)
            wait(0)
            fire_store(g, 0)
            wait(0)

    mesh = plsc.VectorSubcoreMesh(core_axis_name="c", subcore_axis_name="s")
    return functools.partial(
        pl.kernel,
        out_type=jax.ShapeDtypeStruct((EC, width), jnp.float32),
        mesh=mesh,
        scratch_types=[
            pltpu.VMEM((CGWIN, G), jnp.int32),
            pltpu.VMEM((CGWIN, G), jnp.int32),
            pltpu.VMEM((NBUF, G, width), jnp.float32),
            pltpu.SemaphoreType.DMA,
            pltpu.SemaphoreType.DMA,
            pltpu.SemaphoreType.DMA,
        ],
        compiler_params=pltpu.CompilerParams(use_tc_tiling_on_sc=tc_tiling),
    )(gather_kernel)


def _edge_body(es_ref, vec_ref, b0_ref, wl0_ref, w1_ref, b1_ref,
               wa_ref, xtb0_ref, xtw1_ref, xtb1_ref, wb_ref, xob_ref,
               eb_ref, msg_ref, sh_ref):
    vec = vec_ref[...]
    x2 = jnp.sum(vec * vec, axis=-1, keepdims=True)
    ln = jnp.sqrt(x2)
    t0 = _silu(es_ref[...] + ln * wl0_ref[...] + b0_ref[...])
    m1 = _silu(jnp.dot(t0, w1_ref[...],
                       preferred_element_type=jnp.float32) + b1_ref[...])
    pe = jnp.dot(m1, wa_ref[...], preferred_element_type=jnp.float32)
    p1 = _silu(pe[:, :F] + xtb0_ref[...])
    e = _sigm(pe[:, F:F + 1] + eb_ref[0])
    p2 = _silu(jnp.dot(p1, xtw1_ref[...],
                       preferred_element_type=jnp.float32) + xtb1_ref[...])
    phx = jnp.dot(p2, wb_ref[...],
                  preferred_element_type=jnp.float32)[:, :1] + xob_ref[0]
    msg_ref[...] = m1 * e
    sh_ref[...] = phx * vec / (1.0 + ln)


def _edge_mlp(esum, vec, b0, wl0, w1, b1, wa, xtb0, xtw1, xtb1,
              wb, xob, eb):
    vec_spec = pl.BlockSpec((H,), lambda i: (0,))
    mat_spec = pl.BlockSpec((H, H), lambda i: (0, 0))
    one_spec = pl.BlockSpec((1,), lambda i: (0,))
    return pl.pallas_call(
        _edge_body,
        grid=(EC // EBLK,),
        in_specs=[
            pl.BlockSpec((EBLK, F), lambda i: (i, 0)),
            pl.BlockSpec((EBLK, PW), lambda i: (i, 0)),
            vec_spec, vec_spec, mat_spec, vec_spec,
            pl.BlockSpec((H, 2 * H), lambda i: (0, 0)), vec_spec,
            mat_spec, vec_spec, mat_spec, one_spec, one_spec,
        ],
        out_specs=[
            pl.BlockSpec((EBLK, F), lambda i: (i, 0)),
            pl.BlockSpec((EBLK, PW), lambda i: (i, 0)),
        ],
        out_shape=[
            jax.ShapeDtypeStruct((EC, F), jnp.float32),
            jax.ShapeDtypeStruct((EC, PW), jnp.float32),
        ],
    )(esum, vec, b0, wl0, w1, b1, wa, xtb0, xtw1, xtb1, wb, xob, eb)


def _make_scatter(width, tc_tiling):
    nlanes = width // 16

    def scatter_kernel(e0_hbm, e1_hbm, rcv0_hbm, rcv1_hbm, out_hbm,
                       idx_v, rows_v, zbuf_v, acc_sh, sem0, sem1):
        cid = lax.axis_index("c")
        sid = lax.axis_index("s")
        wid = sid * NC + cid
        sems = (sem0, sem1)

        def zrow(i, _):
            r = i // nlanes
            c = i % nlanes
            zbuf_v[r, pl.ds(c * 16, 16)] = jnp.zeros((16,), jnp.float32)
            return 0
        lax.fori_loop(0, CW * nlanes, zrow, 0)

        tbase = sid * RB
        ncp = (RB + jnp.where(sid == NS - 1, N - NS * RB, 0)) // CW

        def zcp(j, _):
            pltpu.sync_copy(zbuf_v, acc_sh.at[pl.ds(tbase + j * CW, CW)])
            return 0
        lax.fori_loop(0, ncp, zcp, 0)

        gstart, ng = _worker_span(wid)
        plsc.subcore_barrier()

        def chunk_pass(e_hbm, rcv2d_hbm):
            d0 = _load_idx(rcv2d_hbm, gstart, idx_v)

            def fire_load(g, b):
                pltpu.async_copy(e_hbm.at[pl.ds((gstart + g) * G, G)],
                                 rows_v.at[b], sems[b])

            def fire_scatter(g, b):
                pltpu.async_copy(rows_v.at[b], acc_sh.at[idx_v.at[d0 + g]],
                                 sems[b], add=True)

            def wait(b):
                pltpu.make_async_copy(e_hbm.at[pl.ds(gstart * G, G)],
                                      rows_v.at[b], sems[b]).wait()

            def blk_body(blk, _):
                g0 = blk * SBUF
                for b in range(SBUF):
                    @pl.when(blk > 0)
                    def _(b=b):
                        wait(b)
                    fire_load(g0 + b, b)
                for b in range(SBUF):
                    wait(b)
                    fire_scatter(g0 + b, b)
                return 0

            lax.fori_loop(0, CSBLK, blk_body, 0)
            for b in range(SBUF):
                wait(b)

            for g in (CSBLK * SBUF, CBASE):
                @pl.when(g < ng)
                def _(g=g):
                    fire_load(g, 0)
                    wait(0)
                    fire_scatter(g, 0)
                    wait(0)

        chunk_pass(e0_hbm, rcv0_hbm)
        chunk_pass(e1_hbm, rcv1_hbm)
        plsc.subcore_barrier()

        def wcp(j, _):
            r0 = tbase + j * CW
            pltpu.sync_copy(acc_sh.at[pl.ds(r0, CW)], zbuf_v)
            pltpu.sync_copy(zbuf_v, out_hbm.at[cid, pl.ds(r0, CW)])
            return 0
        lax.fori_loop(0, ncp, wcp, 0)

    mesh = plsc.VectorSubcoreMesh(core_axis_name="c", subcore_axis_name="s")
    return functools.partial(
        pl.kernel,
        out_type=jax.ShapeDtypeStruct((NC, N, width), jnp.float32),
        mesh=mesh,
        scratch_types=[
            pltpu.VMEM((CGWIN, G), jnp.int32),
            pltpu.VMEM((SBUF, G, width), jnp.float32),
            pltpu.VMEM((CW, width), jnp.float32),
            pltpu.VMEM_SHARED((N, width), jnp.float32),
            pltpu.SemaphoreType.DMA,
            pltpu.SemaphoreType.DMA,
        ],
        compiler_params=pltpu.CompilerParams(use_tc_tiling_on_sc=tc_tiling),
    )(scatter_kernel)


def _node_body(pm_ref, psh_ref, h_ref, pos_ref, hw0m_ref, hw0h_ref, hb0_ref,
               hw1_ref, hb1_ref, hw2_ref, hb2_ref, vout_ref, fout_ref):
    hf = h_ref[...]
    m_i = (pm_ref[0] + pm_ref[1]) / np.float32(np.sqrt(9999.0))
    shift = (psh_ref[0] + psh_ref[1]) / 9999.0
    vout_ref[...] = pos_ref[...] + shift[:, :3]
    t = _silu(jnp.dot(m_i, hw0m_ref[...], preferred_element_type=jnp.float32)
              + jnp.dot(hf, hw0h_ref[...], preferred_element_type=jnp.float32)
              + hb0_ref[...])
    t = _silu(jnp.dot(t, hw1_ref[...], preferred_element_type=jnp.float32)
              + hb1_ref[...])
    fout_ref[...] = (jnp.dot(t, hw2_ref[...],
                             preferred_element_type=jnp.float32)
                     + hb2_ref[...] + hf)


def _node_mlp(pm, psh, h, pos, hw0m, hw0h, hb0, hw1, hb1, hw2, hb2):
    vec_spec = pl.BlockSpec((H,), lambda i: (0,))
    mat_spec = pl.BlockSpec((H, H), lambda i: (0, 0))
    return pl.pallas_call(
        _node_body,
        grid=(N // NB,),
        in_specs=[
            pl.BlockSpec((NC, NB, F), lambda i: (0, i, 0)),
            pl.BlockSpec((NC, NB, PW), lambda i: (0, i, 0)),
            pl.BlockSpec((NB, F), lambda i: (i, 0)),
            pl.BlockSpec((NB, 3), lambda i: (i, 0)),
            mat_spec, mat_spec, vec_spec,
            mat_spec, vec_spec, mat_spec, vec_spec,
        ],
        out_specs=[
            pl.BlockSpec((NB, 3), lambda i: (i, 0)),
            pl.BlockSpec((NB, F), lambda i: (i, 0)),
        ],
        out_shape=[
            jax.ShapeDtypeStruct((N, 3), jnp.float32),
            jax.ShapeDtypeStruct((N, F), jnp.float32),
        ],
    )(pm, psh, h, pos, hw0m, hw0h, hb0, hw1, hb1, hw2, hb2)


def kernel(node_positions, node_features, senders, receivers,
           phi_e_w0, phi_e_b0, phi_e_w1, phi_e_b1,
           phi_xt_w0, phi_xt_b0, phi_xt_w1, phi_xt_b1,
           phi_x_out_w, phi_x_out_b, e_w, e_b,
           phi_h_w0, phi_h_b0, phi_h_w1, phi_h_b1, phi_h_w2, phi_h_b2):
    pos16 = jnp.pad(node_positions, ((0, 0), (0, PW - 3)))
    ws = phi_e_w0[:F]
    wr = phi_e_w0[F:2 * F]
    wl0 = phi_e_w0[2 * F]
    wa = jnp.pad(jnp.concatenate([phi_xt_w0, e_w], axis=1),
                 ((0, 0), (0, H - 1)))
    wb = jnp.pad(phi_x_out_w, ((0, 0), (0, H - 1)))
    hw0m = phi_h_w0[:F]
    hw0h = phi_h_w0[F:]

    snd2d = [jnp.pad(senders[c * EC:(c + 1) * EC].reshape(NGC, G),
                     ((0, CPAD - NGC), (0, 0))) for c in range(NCH)]
    rcv2d = [jnp.pad(receivers[c * EC:(c + 1) * EC].reshape(NGC, G),
                     ((0, CPAD - NGC), (0, 0))) for c in range(NCH)]

    ts, tr, pp, pn = _prep_tables(node_features, ws, wr, pos16)
    gather_f = _make_gather(F, True)
    gather_p = _make_gather(PW, False)
    msgs, shs = [], []
    for c in range(NCH):
        esum = gather_f(ts, tr, snd2d[c], rcv2d[c])
        vec = gather_p(pn, pp, snd2d[c], rcv2d[c])
        msg, sh = _edge_mlp(esum, vec, phi_e_b0, wl0, phi_e_w1, phi_e_b1,
                            wa, phi_xt_b0, phi_xt_w1, phi_xt_b1,
                            wb, phi_x_out_b, e_b)
        msgs.append(msg)
        shs.append(sh)

    pm = _make_scatter(F, True)(msgs[0], msgs[1], rcv2d[0], rcv2d[1])
    psh = _make_scatter(PW, False)(shs[0], shs[1], rcv2d[0], rcv2d[1])
    vectors_out, features_out = _node_mlp(
        pm, psh, node_features, node_positions,
        hw0m, hw0h, phi_h_b0, phi_h_w1, phi_h_b1, phi_h_w2, phi_h_b2)
    return (vectors_out, features_out)
